# baseline (speedup 1.0000x reference)
import functools

import jax
import jax.numpy as jnp
from jax import lax
from jax.experimental import pallas as pl
from jax.experimental.pallas import tpu as pltpu
from jax.experimental.pallas import tpu_sc as plsc


_L = 1024
_D = 1024
_B = 8
_NCG = 4
_CW = _D // _NCG
_NV = _CW // 16
_R = 128
_NCH = _L // _R


def _sc_segmax(x):
    mesh = plsc.VectorSubcoreMesh(core_axis_name="c", subcore_axis_name="s")

    @functools.partial(
        pl.kernel,
        mesh=mesh,
        out_type=jax.ShapeDtypeStruct((_B, _D), jnp.float32),
        scratch_types=[
            pltpu.VMEM((_R, _CW), jnp.float32),
            pltpu.VMEM((_R, _CW), jnp.float32),
            pltpu.VMEM((_CW,), jnp.float32),
            pltpu.SemaphoreType.DMA,
            pltpu.SemaphoreType.DMA,
        ],
    )
    def k(x_hbm, out_hbm, buf0, buf1, ovec, sem0, sem1):
        wid = lax.axis_index("s") * 2 + lax.axis_index("c")
        seg = wid // _NCG
        q = wid % _NCG
        col0 = q * _CW
        row0 = seg * _L
        bufs = (buf0, buf1)
        sems = (sem0, sem1)
        descs = [None, None]

        def start(c):
            b = c % 2
            descs[b] = pltpu.async_copy(
                x_hbm.at[pl.ds(row0 + c * _R, _R), pl.ds(col0, _CW)],
                bufs[b], sems[b])

        start(0)
        acc = tuple(jnp.full((16,), -jnp.inf, jnp.float32) for _ in range(_NV))
        for c in range(_NCH):
            b = c % 2
            if c + 1 < _NCH:
                start(c + 1)
            descs[b].wait()
            buf = bufs[b]

            def body(r, a, buf=buf):
                return tuple(
                    jnp.maximum(a[j], buf[r, pl.ds(j * 16, 16)])
                    for j in range(_NV))

            acc = lax.fori_loop(0, _R, body, acc)
        for j in range(_NV):
            ovec[pl.ds(j * 16, 16)] = acc[j]
        pltpu.sync_copy(ovec, out_hbm.at[seg, pl.ds(col0, _CW)])

    return k(x)


def kernel(x, lengths):
    del lengths
    out = _sc_segmax(x)
    return (out, None)
